# baseline (speedup 1.0000x reference)
import functools

import jax
import jax.numpy as jnp
from jax import lax
from jax.experimental import pallas as pl
from jax.experimental.pallas import tpu as pltpu
from jax.experimental.pallas import tpu_sc as plsc

N = 10000
E = 320000
D1 = 128
D2 = 256

NC = 2
NS = 16
NW = NC * NS
EPT = E // NW
K = 50
NCH = EPT // K
SB = 40
NSB = NCH // SB
NRING = 4
KD = 125
NCHD = EPT // KD
RPT = 624
RPT_LAST = N - 15 * RPT

_mesh = plsc.VectorSubcoreMesh(core_axis_name="c", subcore_axis_name="s")


def _slab_copy(src, dst, s):
    r0 = s * RPT

    @pl.when(s < 15)
    def _():
        pltpu.sync_copy(src.at[pl.ds(r0, RPT)], dst.at[pl.ds(r0, RPT)])

    @pl.when(s == 15)
    def _():
        pltpu.sync_copy(src.at[pl.ds(15 * RPT, RPT_LAST)],
                        dst.at[pl.ds(15 * RPT, RPT_LAST)])


@functools.partial(
    pl.kernel,
    mesh=_mesh,
    out_type=[
        jax.ShapeDtypeStruct((N,), jnp.float32),
        jax.ShapeDtypeStruct((N,), jnp.float32),
    ],
    scratch_types=[
        pltpu.VMEM((NCHD, KD), jnp.int32),
        pltpu.VMEM((128,), jnp.float32),
        pltpu.VMEM_SHARED((N,), jnp.float32),
        pltpu.SemaphoreType.DMA,
    ],
)
def _sc_degree(dst_hbm, zeros_hbm, out0, out1, dst_all, ones_v, acc, sem):
    c = lax.axis_index("c")
    s = lax.axis_index("s")
    w = c * NS + s

    @pl.when(s == 0)
    def _():
        pltpu.sync_copy(zeros_hbm, acc)

    for i in range(8):
        ones_v[pl.ds(i * 16, 16)] = jnp.ones((16,), jnp.float32)
    pltpu.sync_copy(dst_hbm.at[w], dst_all)
    plsc.subcore_barrier()

    ones_src = ones_v.at[pl.ds(0, KD)]
    DEPTH = 8

    def start(j):
        pltpu.async_copy(ones_src, acc.at[dst_all.at[j]], sem, add=True)

    def drain_one():
        pltpu.make_async_copy(ones_src, acc.at[dst_all.at[0]], sem).wait()

    for j in range(DEPTH):
        start(j)

    def body(j, carry):
        drain_one()
        start(j + DEPTH)
        return carry

    lax.fori_loop(0, NCHD - DEPTH, body, 0)
    for _ in range(DEPTH):
        drain_one()
    plsc.subcore_barrier()

    @pl.when(s == 0)
    def _():
        @pl.when(c == 0)
        def _():
            pltpu.sync_copy(acc, out0)

        @pl.when(c == 1)
        def _():
            pltpu.sync_copy(acc, out1)


@functools.partial(
    pl.kernel,
    mesh=_mesh,
    out_type=[
        jax.ShapeDtypeStruct((N, D1), jnp.float32),
        jax.ShapeDtypeStruct((N, D1), jnp.float32),
    ],
    scratch_types=[
        pltpu.VMEM((SB, K), jnp.int32),
        pltpu.VMEM((SB, K), jnp.int32),
        pltpu.VMEM((SB, K), jnp.int32),
        pltpu.VMEM((SB, K), jnp.int32),
        pltpu.VMEM((K, D1), jnp.float32),
        pltpu.VMEM((K, D1), jnp.float32),
        pltpu.VMEM((K, D1), jnp.float32),
        pltpu.VMEM((K, D1), jnp.float32),
        pltpu.VMEM_SHARED((N, D1), jnp.float32),
        pltpu.SemaphoreType.DMA,
        pltpu.SemaphoreType.DMA,
        pltpu.SemaphoreType.DMA,
        pltpu.SemaphoreType.DMA,
        pltpu.SemaphoreType.DMA,
        pltpu.SemaphoreType.DMA,
        pltpu.SemaphoreType.DMA,
        pltpu.SemaphoreType.DMA,
        pltpu.SemaphoreType.DMA,
    ],
)
def _sc_scatter(g_hbm, src_hbm, dst_hbm, zeros_hbm, out0, out1,
                isrc0, idst0, isrc1, idst1, r0b, r1b, r2b, r3b, acc,
                gsem0, gsem1, gsem2, gsem3, ssem0, ssem1, ssem2, ssem3, isem):
    c = lax.axis_index("c")
    s = lax.axis_index("s")
    w = c * NS + s
    rows = [r0b, r1b, r2b, r3b]
    gsem = [gsem0, gsem1, gsem2, gsem3]
    ssem = [ssem0, ssem1, ssem2, ssem3]
    ibufs = [(isrc0, idst0), (isrc1, idst1)]

    _slab_copy(zeros_hbm, acc, s)
    pltpu.sync_copy(src_hbm.at[w, 0], isrc0)
    pltpu.sync_copy(dst_hbm.at[w, 0], idst0)
    plsc.subcore_barrier()

    for si in range(NSB):
        s_src, s_dst = ibufs[si & 1]
        if si + 1 < NSB:
            n_src, n_dst = ibufs[1 - (si & 1)]
            pltpu.async_copy(src_hbm.at[w, si + 1], n_src, isem)
            pltpu.async_copy(dst_hbm.at[w, si + 1], n_dst, isem)

        def gather_start(j, b):
            pltpu.async_copy(g_hbm.at[s_src.at[j]], rows[b], gsem[b])

        def gather_wait(b):
            pltpu.make_async_copy(g_hbm.at[s_src.at[0]], rows[b],
                                  gsem[b]).wait()

        def scatter_start(j, b):
            pltpu.async_copy(rows[b], acc.at[s_dst.at[j]], ssem[b], add=True)

        def scatter_wait(b):
            pltpu.make_async_copy(rows[b], acc.at[s_dst.at[0]],
                                  ssem[b]).wait()

        gather_start(0, 0)
        gather_start(1, 1)

        def body(t, carry):
            j0 = 4 * t
            for b in range(4):
                j = j0 + b
                gather_wait(b)
                scatter_start(j, b)
                jj = j + 2

                @pl.when(jnp.logical_and(jj < SB, jj - 4 >= 0))
                def _(b2=(b + 2) % 4, jj=jj):
                    scatter_wait(b2)
                    gather_start(jj, b2)

                @pl.when(jnp.logical_and(jj < SB, jj - 4 < 0))
                def _(b2=(b + 2) % 4, jj=jj):
                    gather_start(jj, b2)
            return carry

        lax.fori_loop(0, SB // 4, body, 0)
        for b in range(4):
            scatter_wait(b)
        if si + 1 < NSB:
            pltpu.make_async_copy(src_hbm.at[w, 0], n_src, isem).wait()
            pltpu.make_async_copy(dst_hbm.at[w, 0], n_dst, isem).wait()
    plsc.subcore_barrier()

    @pl.when(c == 0)
    def _():
        _slab_copy(acc, out0, s)

    @pl.when(c == 1)
    def _():
        _slab_copy(acc, out1, s)


R = 1000
GRID = N // R


def _row_spec(d):
    return pl.BlockSpec((R, d), lambda i: (i, 0))


def _full_spec(a, b):
    return pl.BlockSpec((a, b), lambda i: (0, 0))


def _ln(t, g, b):
    mu = jnp.mean(t, axis=-1, keepdims=True)
    var = jnp.mean((t - mu) ** 2, axis=-1, keepdims=True)
    return (t - mu) / jnp.sqrt(var + 1e-5) * g + b


def _tc1a_body(x_ref, w1_ref, fw_ref, fb_ref, lg_ref, lb_ref,
               h1_ref, f1_ref):
    x = x_ref[...]
    h1_ref[...] = jnp.dot(x, w1_ref[...], preferred_element_type=jnp.float32)
    f = jnp.dot(x, fw_ref[...], preferred_element_type=jnp.float32) + fb_ref[...]
    f1_ref[...] = jnp.maximum(_ln(f, lg_ref[...], lb_ref[...]), 0.0)


_tc1a = pl.pallas_call(
    _tc1a_body,
    grid=(GRID,),
    in_specs=[
        _row_spec(D1), _full_spec(D1, D1), _full_spec(D1, D1),
        _full_spec(1, D1), _full_spec(1, D1), _full_spec(1, D1),
    ],
    out_specs=[_row_spec(D1), _row_spec(D1)],
    out_shape=[
        jax.ShapeDtypeStruct((N, D1), jnp.float32),
        jax.ShapeDtypeStruct((N, D1), jnp.float32),
    ],
)


def _tc1b_body(h1_ref, d0_ref, d1_ref, g1_ref, dinv_ref):
    deg = d0_ref[...] + d1_ref[...] + 1.0
    dinv = lax.rsqrt(deg)
    dinv_ref[...] = dinv
    g1_ref[...] = h1_ref[...] * dinv


_tc1b = pl.pallas_call(
    _tc1b_body,
    grid=(GRID,),
    in_specs=[_row_spec(D1), _row_spec(1), _row_spec(1)],
    out_specs=[_row_spec(D1), _row_spec(1)],
    out_shape=[
        jax.ShapeDtypeStruct((N, D1), jnp.float32),
        jax.ShapeDtypeStruct((N, 1), jnp.float32),
    ],
)


def _tc2_body(a_ref, b_ref, g1_ref, dinv_ref, f1_ref, w2a_ref, w2b_ref, g2_ref):
    dinv = dinv_ref[...]
    x1 = jnp.maximum(dinv * (a_ref[...] + b_ref[...] + g1_ref[...]), 0.0)
    h2 = (jnp.dot(x1, w2a_ref[...], preferred_element_type=jnp.float32)
          + jnp.dot(f1_ref[...], w2b_ref[...], preferred_element_type=jnp.float32))
    g2_ref[...] = h2 * dinv


_tc2 = pl.pallas_call(
    _tc2_body,
    grid=(GRID,),
    in_specs=[
        _row_spec(D1), _row_spec(D1), _row_spec(D1), _row_spec(1),
        _row_spec(D1), _full_spec(D1, D1), _full_spec(D1, D1),
    ],
    out_specs=[_row_spec(D1)],
    out_shape=[jax.ShapeDtypeStruct((N, D1), jnp.float32)],
)


def _tc3_body(a_ref, b_ref, g2_ref, dinv_ref, b2_ref, lg_ref, lb_ref, w3_ref,
              g3_ref):
    dinv = dinv_ref[...]
    t = dinv * (a_ref[...] + b_ref[...] + g2_ref[...]) + b2_ref[...]
    x2 = jnp.maximum(_ln(t, lg_ref[...], lb_ref[...]), 0.0)
    g3_ref[...] = jnp.dot(x2, w3_ref[...], preferred_element_type=jnp.float32) * dinv


_tc3 = pl.pallas_call(
    _tc3_body,
    grid=(GRID,),
    in_specs=[
        _row_spec(D1), _row_spec(D1), _row_spec(D1), _row_spec(1),
        _full_spec(1, D1), _full_spec(1, D1), _full_spec(1, D1),
        _full_spec(D1, D1),
    ],
    out_specs=[_row_spec(D1)],
    out_shape=[jax.ShapeDtypeStruct((N, D1), jnp.float32)],
)


def _tc4_body(a_ref, b_ref, g3_ref, dinv_ref, b3_ref, lg_ref, lb_ref, x3_ref):
    dinv = dinv_ref[...]
    t = dinv * (a_ref[...] + b_ref[...] + g3_ref[...]) + b3_ref[...]
    x3_ref[...] = _ln(t, lg_ref[...], lb_ref[...])


_tc4 = pl.pallas_call(
    _tc4_body,
    grid=(GRID,),
    in_specs=[
        _row_spec(D1), _row_spec(D1), _row_spec(D1), _row_spec(1),
        _full_spec(1, D1), _full_spec(1, D1), _full_spec(1, D1),
    ],
    out_specs=[_row_spec(D1)],
    out_shape=[jax.ShapeDtypeStruct((N, D1), jnp.float32)],
)


def kernel(x, edge_index, W1, fc1_W, fc1_b, ln1_g, ln1_b, W2, b2, ln2_g, ln2_b,
           W3, b3, ln3_g, ln3_b):
    src3 = edge_index[0].reshape(NW, NSB, SB, K)
    dst3 = edge_index[1].reshape(NW, NSB, SB, K)
    dst3d = edge_index[1].reshape(NW, NCHD, KD)
    zeros1 = jnp.zeros((N,), jnp.float32)
    zeros2 = jnp.zeros((N, D1), jnp.float32)

    d0, d1 = _sc_degree(dst3d, zeros1)

    h1, f1 = _tc1a(x, W1, fc1_W, fc1_b.reshape(1, D1), ln1_g.reshape(1, D1),
                   ln1_b.reshape(1, D1))
    g1, dinv = _tc1b(h1, d0.reshape(N, 1), d1.reshape(N, 1))

    a1a, a1b = _sc_scatter(g1, src3, dst3, zeros2)
    (g2,) = _tc2(a1a, a1b, g1, dinv, f1, W2[:D1], W2[D1:])

    a2a, a2b = _sc_scatter(g2, src3, dst3, zeros2)
    (g3,) = _tc3(a2a, a2b, g2, dinv, b2.reshape(1, D1),
                 ln2_g.reshape(1, D1), ln2_b.reshape(1, D1), W3)

    a3a, a3b = _sc_scatter(g3, src3, dst3, zeros2)
    (x3,) = _tc4(a3a, a3b, g3, dinv, b3.reshape(1, D1),
                 ln3_g.reshape(1, D1), ln3_b.reshape(1, D1))
    return x3


# speedup vs baseline: 1.0139x; 1.0139x over previous
import functools

import jax
import jax.numpy as jnp
from jax import lax
from jax.experimental import pallas as pl
from jax.experimental.pallas import tpu as pltpu
from jax.experimental.pallas import tpu_sc as plsc

N = 10000
E = 320000
D1 = 128
D2 = 256

NC = 2
NS = 16
NW = NC * NS
EPT = E // NW
K = 50
NCH = EPT // K
SB = 40
NSB = NCH // SB
NRING = 4
KD = 125
NCHD = EPT // KD
RPT = 624
RPT_LAST = N - 15 * RPT

_mesh = plsc.VectorSubcoreMesh(core_axis_name="c", subcore_axis_name="s")


def _slab_copy(src, dst, s):
    r0 = s * RPT

    @pl.when(s < 15)
    def _():
        pltpu.sync_copy(src.at[pl.ds(r0, RPT)], dst.at[pl.ds(r0, RPT)])

    @pl.when(s == 15)
    def _():
        pltpu.sync_copy(src.at[pl.ds(15 * RPT, RPT_LAST)],
                        dst.at[pl.ds(15 * RPT, RPT_LAST)])


@functools.partial(
    pl.kernel,
    mesh=_mesh,
    out_type=[
        jax.ShapeDtypeStruct((N,), jnp.float32),
        jax.ShapeDtypeStruct((N,), jnp.float32),
    ],
    scratch_types=[
        pltpu.VMEM((NCHD, KD), jnp.int32),
        pltpu.VMEM((128,), jnp.float32),
        pltpu.VMEM_SHARED((N,), jnp.float32),
        pltpu.SemaphoreType.DMA,
    ],
)
def _sc_degree(dst_hbm, zeros_hbm, out0, out1, dst_all, ones_v, acc, sem):
    c = lax.axis_index("c")
    s = lax.axis_index("s")
    w = c * NS + s

    @pl.when(s == 0)
    def _():
        pltpu.sync_copy(zeros_hbm, acc)

    for i in range(8):
        ones_v[pl.ds(i * 16, 16)] = jnp.ones((16,), jnp.float32)
    pltpu.sync_copy(dst_hbm.at[w], dst_all)
    plsc.subcore_barrier()

    ones_src = ones_v.at[pl.ds(0, KD)]
    DEPTH = 8

    def start(j):
        pltpu.async_copy(ones_src, acc.at[dst_all.at[j]], sem, add=True)

    def drain_one():
        pltpu.make_async_copy(ones_src, acc.at[dst_all.at[0]], sem).wait()

    for j in range(DEPTH):
        start(j)

    def body(j, carry):
        drain_one()
        start(j + DEPTH)
        return carry

    lax.fori_loop(0, NCHD - DEPTH, body, 0)
    for _ in range(DEPTH):
        drain_one()
    plsc.subcore_barrier()

    @pl.when(s == 0)
    def _():
        @pl.when(c == 0)
        def _():
            pltpu.sync_copy(acc, out0)

        @pl.when(c == 1)
        def _():
            pltpu.sync_copy(acc, out1)


@functools.partial(
    pl.kernel,
    mesh=_mesh,
    out_type=[
        jax.ShapeDtypeStruct((N, D1), jnp.float32),
        jax.ShapeDtypeStruct((N, D1), jnp.float32),
    ],
    scratch_types=[
        pltpu.VMEM((SB, K), jnp.int32),
        pltpu.VMEM((SB, K), jnp.int32),
        pltpu.VMEM((SB, K), jnp.int32),
        pltpu.VMEM((SB, K), jnp.int32),
        pltpu.VMEM((K, D1), jnp.float32),
        pltpu.VMEM((K, D1), jnp.float32),
        pltpu.VMEM((K, D1), jnp.float32),
        pltpu.VMEM((K, D1), jnp.float32),
        pltpu.VMEM_SHARED((N, D1), jnp.float32),
        pltpu.SemaphoreType.DMA,
        pltpu.SemaphoreType.DMA,
        pltpu.SemaphoreType.DMA,
        pltpu.SemaphoreType.DMA,
        pltpu.SemaphoreType.DMA,
        pltpu.SemaphoreType.DMA,
        pltpu.SemaphoreType.DMA,
        pltpu.SemaphoreType.DMA,
        pltpu.SemaphoreType.DMA,
    ],
)
def _sc_scatter(g_hbm, src_hbm, dst_hbm, zeros_hbm, out0, out1,
                isrc0, idst0, isrc1, idst1, r0b, r1b, r2b, r3b, acc,
                gsem0, gsem1, gsem2, gsem3, ssem0, ssem1, ssem2, ssem3, isem):
    c = lax.axis_index("c")
    s = lax.axis_index("s")
    w = c * NS + s
    rows = [r0b, r1b, r2b, r3b]
    gsem = [gsem0, gsem1, gsem2, gsem3]
    ssem = [ssem0, ssem1, ssem2, ssem3]
    ibufs = [(isrc0, idst0), (isrc1, idst1)]

    _slab_copy(zeros_hbm, acc, s)
    pltpu.sync_copy(src_hbm.at[w, 0], isrc0)
    pltpu.sync_copy(dst_hbm.at[w, 0], idst0)
    plsc.subcore_barrier()

    for si in range(NSB):
        s_src, s_dst = ibufs[si & 1]
        if si + 1 < NSB:
            n_src, n_dst = ibufs[1 - (si & 1)]
            pltpu.async_copy(src_hbm.at[w, si + 1], n_src, isem)
            pltpu.async_copy(dst_hbm.at[w, si + 1], n_dst, isem)

        def gather_start(j, b):
            pltpu.async_copy(g_hbm.at[s_src.at[j]], rows[b], gsem[b])

        def gather_wait(b):
            pltpu.make_async_copy(g_hbm.at[s_src.at[0]], rows[b],
                                  gsem[b]).wait()

        def scatter_start(j, b):
            pltpu.async_copy(rows[b], acc.at[s_dst.at[j]], ssem[b], add=True)

        def scatter_wait(b):
            pltpu.make_async_copy(rows[b], acc.at[s_dst.at[0]],
                                  ssem[b]).wait()

        gather_start(0, 0)
        gather_start(1, 1)

        def body(t, carry):
            j0 = 4 * t
            for b in range(4):
                j = j0 + b
                gather_wait(b)
                scatter_start(j, b)
                jj = j + 2

                @pl.when(jnp.logical_and(jj < SB, jj - 4 >= 0))
                def _(b2=(b + 2) % 4, jj=jj):
                    scatter_wait(b2)
                    gather_start(jj, b2)

                @pl.when(jnp.logical_and(jj < SB, jj - 4 < 0))
                def _(b2=(b + 2) % 4, jj=jj):
                    gather_start(jj, b2)
            return carry

        lax.fori_loop(0, SB // 4, body, 0)
        for b in range(4):
            scatter_wait(b)
        if si + 1 < NSB:
            pltpu.make_async_copy(src_hbm.at[w, 0], n_src, isem).wait()
            pltpu.make_async_copy(dst_hbm.at[w, 0], n_dst, isem).wait()
    plsc.subcore_barrier()

    @pl.when(c == 0)
    def _():
        _slab_copy(acc, out0, s)

    @pl.when(c == 1)
    def _():
        _slab_copy(acc, out1, s)


R = 1000
GRID = N // R


def _row_spec(d):
    return pl.BlockSpec((R, d), lambda i: (i, 0))


def _full_spec(a, b):
    return pl.BlockSpec((a, b), lambda i: (0, 0))


def _ln(t, g, b):
    mu = jnp.mean(t, axis=-1, keepdims=True)
    var = jnp.mean((t - mu) ** 2, axis=-1, keepdims=True)
    return (t - mu) / jnp.sqrt(var + 1e-5) * g + b


def _tc1_body(x_ref, w1_ref, fw_ref, fb_ref, lg_ref, lb_ref, d0_ref, d1_ref,
              g1_ref, f1_ref, dinv_ref):
    x = x_ref[...]
    deg = d0_ref[...] + d1_ref[...] + 1.0
    dinv = lax.rsqrt(deg)
    dinv_ref[...] = dinv
    h1 = jnp.dot(x, w1_ref[...], preferred_element_type=jnp.float32)
    g1_ref[...] = h1 * dinv
    f = jnp.dot(x, fw_ref[...], preferred_element_type=jnp.float32) + fb_ref[...]
    f1_ref[...] = jnp.maximum(_ln(f, lg_ref[...], lb_ref[...]), 0.0)


_tc1 = pl.pallas_call(
    _tc1_body,
    grid=(GRID,),
    in_specs=[
        _row_spec(D1), _full_spec(D1, D1), _full_spec(D1, D1),
        _full_spec(1, D1), _full_spec(1, D1), _full_spec(1, D1),
        _row_spec(1), _row_spec(1),
    ],
    out_specs=[_row_spec(D1), _row_spec(D1), _row_spec(1)],
    out_shape=[
        jax.ShapeDtypeStruct((N, D1), jnp.float32),
        jax.ShapeDtypeStruct((N, D1), jnp.float32),
        jax.ShapeDtypeStruct((N, 1), jnp.float32),
    ],
)


def _tc2_body(a_ref, b_ref, g1_ref, dinv_ref, f1_ref, w2a_ref, w2b_ref, g2_ref):
    dinv = dinv_ref[...]
    x1 = jnp.maximum(dinv * (a_ref[...] + b_ref[...] + g1_ref[...]), 0.0)
    h2 = (jnp.dot(x1, w2a_ref[...], preferred_element_type=jnp.float32)
          + jnp.dot(f1_ref[...], w2b_ref[...], preferred_element_type=jnp.float32))
    g2_ref[...] = h2 * dinv


_tc2 = pl.pallas_call(
    _tc2_body,
    grid=(GRID,),
    in_specs=[
        _row_spec(D1), _row_spec(D1), _row_spec(D1), _row_spec(1),
        _row_spec(D1), _full_spec(D1, D1), _full_spec(D1, D1),
    ],
    out_specs=[_row_spec(D1)],
    out_shape=[jax.ShapeDtypeStruct((N, D1), jnp.float32)],
)


def _tc3_body(a_ref, b_ref, g2_ref, dinv_ref, b2_ref, lg_ref, lb_ref, w3_ref,
              g3_ref):
    dinv = dinv_ref[...]
    t = dinv * (a_ref[...] + b_ref[...] + g2_ref[...]) + b2_ref[...]
    x2 = jnp.maximum(_ln(t, lg_ref[...], lb_ref[...]), 0.0)
    g3_ref[...] = jnp.dot(x2, w3_ref[...], preferred_element_type=jnp.float32) * dinv


_tc3 = pl.pallas_call(
    _tc3_body,
    grid=(GRID,),
    in_specs=[
        _row_spec(D1), _row_spec(D1), _row_spec(D1), _row_spec(1),
        _full_spec(1, D1), _full_spec(1, D1), _full_spec(1, D1),
        _full_spec(D1, D1),
    ],
    out_specs=[_row_spec(D1)],
    out_shape=[jax.ShapeDtypeStruct((N, D1), jnp.float32)],
)


def _tc4_body(a_ref, b_ref, g3_ref, dinv_ref, b3_ref, lg_ref, lb_ref, x3_ref):
    dinv = dinv_ref[...]
    t = dinv * (a_ref[...] + b_ref[...] + g3_ref[...]) + b3_ref[...]
    x3_ref[...] = _ln(t, lg_ref[...], lb_ref[...])


_tc4 = pl.pallas_call(
    _tc4_body,
    grid=(GRID,),
    in_specs=[
        _row_spec(D1), _row_spec(D1), _row_spec(D1), _row_spec(1),
        _full_spec(1, D1), _full_spec(1, D1), _full_spec(1, D1),
    ],
    out_specs=[_row_spec(D1)],
    out_shape=[jax.ShapeDtypeStruct((N, D1), jnp.float32)],
)


def kernel(x, edge_index, W1, fc1_W, fc1_b, ln1_g, ln1_b, W2, b2, ln2_g, ln2_b,
           W3, b3, ln3_g, ln3_b):
    src3 = edge_index[0].reshape(NW, NSB, SB, K)
    dst3 = edge_index[1].reshape(NW, NSB, SB, K)
    dst3d = edge_index[1].reshape(NW, NCHD, KD)
    zeros1 = jnp.zeros((N,), jnp.float32)
    zeros2 = jnp.zeros((N, D1), jnp.float32)

    d0, d1 = _sc_degree(dst3d, zeros1)

    g1, f1, dinv = _tc1(
        x, W1, fc1_W, fc1_b.reshape(1, D1), ln1_g.reshape(1, D1),
        ln1_b.reshape(1, D1), d0.reshape(N, 1), d1.reshape(N, 1))

    a1a, a1b = _sc_scatter(g1, src3, dst3, zeros2)
    (g2,) = _tc2(a1a, a1b, g1, dinv, f1, W2[:D1], W2[D1:])

    a2a, a2b = _sc_scatter(g2, src3, dst3, zeros2)
    (g3,) = _tc3(a2a, a2b, g2, dinv, b2.reshape(1, D1),
                 ln2_g.reshape(1, D1), ln2_b.reshape(1, D1), W3)

    a3a, a3b = _sc_scatter(g3, src3, dst3, zeros2)
    (x3,) = _tc4(a3a, a3b, g3, dinv, b3.reshape(1, D1),
                 ln3_g.reshape(1, D1), ln3_b.reshape(1, D1))
    return x3


# speedup vs baseline: 1.0324x; 1.0182x over previous
import functools

import jax
import jax.numpy as jnp
from jax import lax
from jax.experimental import pallas as pl
from jax.experimental.pallas import tpu as pltpu
from jax.experimental.pallas import tpu_sc as plsc

N = 10000
E = 320000
D1 = 128
D2 = 256

NC = 2
NS = 16
NW = NC * NS
EPT = E // NW
K = 50
NCH = EPT // K
SB = 40
NSB = NCH // SB
NRING = 4
KD = 125
NCHD = EPT // KD
RPT = 624
RPT_LAST = N - 15 * RPT

_mesh = plsc.VectorSubcoreMesh(core_axis_name="c", subcore_axis_name="s")


def _slab_copy(src, dst, s):
    r0 = s * RPT

    @pl.when(s < 15)
    def _():
        pltpu.sync_copy(src.at[pl.ds(r0, RPT)], dst.at[pl.ds(r0, RPT)])

    @pl.when(s == 15)
    def _():
        pltpu.sync_copy(src.at[pl.ds(15 * RPT, RPT_LAST)],
                        dst.at[pl.ds(15 * RPT, RPT_LAST)])


@functools.partial(
    pl.kernel,
    mesh=_mesh,
    out_type=[
        jax.ShapeDtypeStruct((N,), jnp.float32),
        jax.ShapeDtypeStruct((N,), jnp.float32),
    ],
    scratch_types=[
        pltpu.VMEM((NCHD, KD), jnp.int32),
        pltpu.VMEM((128,), jnp.float32),
        pltpu.VMEM_SHARED((N,), jnp.float32),
        pltpu.SemaphoreType.DMA,
    ],
)
def _sc_degree(dst_hbm, zeros_hbm, out0, out1, dst_all, ones_v, acc, sem):
    c = lax.axis_index("c")
    s = lax.axis_index("s")
    w = c * NS + s

    @pl.when(s == 0)
    def _():
        pltpu.sync_copy(zeros_hbm, acc)

    for i in range(8):
        ones_v[pl.ds(i * 16, 16)] = jnp.ones((16,), jnp.float32)
    pltpu.sync_copy(dst_hbm.at[w], dst_all)
    plsc.subcore_barrier()

    ones_src = ones_v.at[pl.ds(0, KD)]
    DEPTH = 8

    def start(j):
        pltpu.async_copy(ones_src, acc.at[dst_all.at[j]], sem, add=True)

    def drain_one():
        pltpu.make_async_copy(ones_src, acc.at[dst_all.at[0]], sem).wait()

    for j in range(DEPTH):
        start(j)

    def body(j, carry):
        drain_one()
        start(j + DEPTH)
        return carry

    lax.fori_loop(0, NCHD - DEPTH, body, 0)
    for _ in range(DEPTH):
        drain_one()
    plsc.subcore_barrier()

    @pl.when(s == 0)
    def _():
        @pl.when(c == 0)
        def _():
            pltpu.sync_copy(acc, out0)

        @pl.when(c == 1)
        def _():
            pltpu.sync_copy(acc, out1)


@functools.partial(
    pl.kernel,
    mesh=_mesh,
    out_type=[
        jax.ShapeDtypeStruct((N, D1), jnp.float32),
        jax.ShapeDtypeStruct((N, D1), jnp.float32),
    ],
    scratch_types=[
        pltpu.VMEM((SB, K), jnp.int32),
        pltpu.VMEM((SB, K), jnp.int32),
        pltpu.VMEM((SB, K), jnp.int32),
        pltpu.VMEM((SB, K), jnp.int32),
        pltpu.VMEM((K, D1), jnp.float32),
        pltpu.VMEM((K, D1), jnp.float32),
        pltpu.VMEM((K, D1), jnp.float32),
        pltpu.VMEM((K, D1), jnp.float32),
        pltpu.VMEM_SHARED((N, D1), jnp.float32),
        pltpu.SemaphoreType.DMA,
        pltpu.SemaphoreType.DMA,
        pltpu.SemaphoreType.DMA,
        pltpu.SemaphoreType.DMA,
        pltpu.SemaphoreType.DMA,
        pltpu.SemaphoreType.DMA,
        pltpu.SemaphoreType.DMA,
        pltpu.SemaphoreType.DMA,
        pltpu.SemaphoreType.DMA,
    ],
)
def _sc_scatter(g_hbm, src_hbm, dst_hbm, zeros_hbm, out0, out1,
                isrc0, idst0, isrc1, idst1, r0b, r1b, r2b, r3b, acc,
                gsem0, gsem1, gsem2, gsem3, ssem0, ssem1, ssem2, ssem3, isem):
    c = lax.axis_index("c")
    s = lax.axis_index("s")
    w = c * NS + s
    rows = [r0b, r1b, r2b, r3b]
    gsem = [gsem0, gsem1, gsem2, gsem3]
    ssem = [ssem0, ssem1, ssem2, ssem3]
    ibufs = [(isrc0, idst0), (isrc1, idst1)]

    _slab_copy(zeros_hbm, acc, s)
    pltpu.sync_copy(src_hbm.at[w, 0], isrc0)
    pltpu.sync_copy(dst_hbm.at[w, 0], idst0)
    plsc.subcore_barrier()

    for si in range(NSB):
        s_src, s_dst = ibufs[si & 1]
        if si + 1 < NSB:
            n_src, n_dst = ibufs[1 - (si & 1)]
            pltpu.async_copy(src_hbm.at[w, si + 1], n_src, isem)
            pltpu.async_copy(dst_hbm.at[w, si + 1], n_dst, isem)

        def gather_start(j, b):
            pltpu.async_copy(g_hbm.at[s_src.at[j]], rows[b], gsem[b])

        def gather_wait(b):
            pltpu.make_async_copy(g_hbm.at[s_src.at[0]], rows[b],
                                  gsem[b]).wait()

        def scatter_start(j, b):
            pltpu.async_copy(rows[b], acc.at[s_dst.at[j]], ssem[b], add=True)

        def scatter_wait(b):
            pltpu.make_async_copy(rows[b], acc.at[s_dst.at[0]],
                                  ssem[b]).wait()

        gather_start(0, 0)
        gather_start(1, 1)

        def body(t, carry):
            j0 = 4 * t
            for b in range(4):
                j = j0 + b
                gather_wait(b)
                scatter_start(j, b)
                jj = j + 2

                @pl.when(jnp.logical_and(jj < SB, jj - 4 >= 0))
                def _(b2=(b + 2) % 4, jj=jj):
                    scatter_wait(b2)
                    gather_start(jj, b2)

                @pl.when(jnp.logical_and(jj < SB, jj - 4 < 0))
                def _(b2=(b + 2) % 4, jj=jj):
                    gather_start(jj, b2)
            return carry

        lax.fori_loop(0, SB // 4, body, 0)
        for b in range(4):
            scatter_wait(b)
        if si + 1 < NSB:
            pltpu.make_async_copy(src_hbm.at[w, 0], n_src, isem).wait()
            pltpu.make_async_copy(dst_hbm.at[w, 0], n_dst, isem).wait()
    plsc.subcore_barrier()

    @pl.when(c == 0)
    def _():
        _slab_copy(acc, out0, s)

    @pl.when(c == 1)
    def _():
        _slab_copy(acc, out1, s)


R = 2000
GRID = N // R


def _row_spec(d):
    return pl.BlockSpec((R, d), lambda i: (i, 0))


def _full_spec(a, b):
    return pl.BlockSpec((a, b), lambda i: (0, 0))


def _ln(t, g, b):
    mu = jnp.mean(t, axis=-1, keepdims=True)
    var = jnp.mean((t - mu) ** 2, axis=-1, keepdims=True)
    return (t - mu) / jnp.sqrt(var + 1e-5) * g + b


def _tc1_body(x_ref, w1_ref, fw_ref, fb_ref, lg_ref, lb_ref, d0_ref, d1_ref,
              g1_ref, f1_ref, dinv_ref):
    x = x_ref[...]
    deg = d0_ref[...] + d1_ref[...] + 1.0
    dinv = lax.rsqrt(deg)
    dinv_ref[...] = dinv
    h1 = jnp.dot(x, w1_ref[...], preferred_element_type=jnp.float32)
    g1_ref[...] = h1 * dinv
    f = jnp.dot(x, fw_ref[...], preferred_element_type=jnp.float32) + fb_ref[...]
    f1_ref[...] = jnp.maximum(_ln(f, lg_ref[...], lb_ref[...]), 0.0)


_tc1 = pl.pallas_call(
    _tc1_body,
    grid=(GRID,),
    in_specs=[
        _row_spec(D1), _full_spec(D1, D1), _full_spec(D1, D1),
        _full_spec(1, D1), _full_spec(1, D1), _full_spec(1, D1),
        _row_spec(1), _row_spec(1),
    ],
    out_specs=[_row_spec(D1), _row_spec(D1), _row_spec(1)],
    out_shape=[
        jax.ShapeDtypeStruct((N, D1), jnp.float32),
        jax.ShapeDtypeStruct((N, D1), jnp.float32),
        jax.ShapeDtypeStruct((N, 1), jnp.float32),
    ],
)


def _tc2_body(a_ref, b_ref, g1_ref, dinv_ref, f1_ref, w2a_ref, w2b_ref, g2_ref):
    dinv = dinv_ref[...]
    x1 = jnp.maximum(dinv * (a_ref[...] + b_ref[...] + g1_ref[...]), 0.0)
    h2 = (jnp.dot(x1, w2a_ref[...], preferred_element_type=jnp.float32)
          + jnp.dot(f1_ref[...], w2b_ref[...], preferred_element_type=jnp.float32))
    g2_ref[...] = h2 * dinv


_tc2 = pl.pallas_call(
    _tc2_body,
    grid=(GRID,),
    in_specs=[
        _row_spec(D1), _row_spec(D1), _row_spec(D1), _row_spec(1),
        _row_spec(D1), _full_spec(D1, D1), _full_spec(D1, D1),
    ],
    out_specs=[_row_spec(D1)],
    out_shape=[jax.ShapeDtypeStruct((N, D1), jnp.float32)],
)


def _tc3_body(a_ref, b_ref, g2_ref, dinv_ref, b2_ref, lg_ref, lb_ref, w3_ref,
              g3_ref):
    dinv = dinv_ref[...]
    t = dinv * (a_ref[...] + b_ref[...] + g2_ref[...]) + b2_ref[...]
    x2 = jnp.maximum(_ln(t, lg_ref[...], lb_ref[...]), 0.0)
    g3_ref[...] = jnp.dot(x2, w3_ref[...], preferred_element_type=jnp.float32) * dinv


_tc3 = pl.pallas_call(
    _tc3_body,
    grid=(GRID,),
    in_specs=[
        _row_spec(D1), _row_spec(D1), _row_spec(D1), _row_spec(1),
        _full_spec(1, D1), _full_spec(1, D1), _full_spec(1, D1),
        _full_spec(D1, D1),
    ],
    out_specs=[_row_spec(D1)],
    out_shape=[jax.ShapeDtypeStruct((N, D1), jnp.float32)],
)


def _tc4_body(a_ref, b_ref, g3_ref, dinv_ref, b3_ref, lg_ref, lb_ref, x3_ref):
    dinv = dinv_ref[...]
    t = dinv * (a_ref[...] + b_ref[...] + g3_ref[...]) + b3_ref[...]
    x3_ref[...] = _ln(t, lg_ref[...], lb_ref[...])


_tc4 = pl.pallas_call(
    _tc4_body,
    grid=(GRID,),
    in_specs=[
        _row_spec(D1), _row_spec(D1), _row_spec(D1), _row_spec(1),
        _full_spec(1, D1), _full_spec(1, D1), _full_spec(1, D1),
    ],
    out_specs=[_row_spec(D1)],
    out_shape=[jax.ShapeDtypeStruct((N, D1), jnp.float32)],
)


def kernel(x, edge_index, W1, fc1_W, fc1_b, ln1_g, ln1_b, W2, b2, ln2_g, ln2_b,
           W3, b3, ln3_g, ln3_b):
    src3 = edge_index[0].reshape(NW, NSB, SB, K)
    dst3 = edge_index[1].reshape(NW, NSB, SB, K)
    dst3d = edge_index[1].reshape(NW, NCHD, KD)
    zeros1 = jnp.zeros((N,), jnp.float32)
    zeros2 = jnp.zeros((N, D1), jnp.float32)

    d0, d1 = _sc_degree(dst3d, zeros1)

    g1, f1, dinv = _tc1(
        x, W1, fc1_W, fc1_b.reshape(1, D1), ln1_g.reshape(1, D1),
        ln1_b.reshape(1, D1), d0.reshape(N, 1), d1.reshape(N, 1))

    a1a, a1b = _sc_scatter(g1, src3, dst3, zeros2)
    (g2,) = _tc2(a1a, a1b, g1, dinv, f1, W2[:D1], W2[D1:])

    a2a, a2b = _sc_scatter(g2, src3, dst3, zeros2)
    (g3,) = _tc3(a2a, a2b, g2, dinv, b2.reshape(1, D1),
                 ln2_g.reshape(1, D1), ln2_b.reshape(1, D1), W3)

    a3a, a3b = _sc_scatter(g3, src3, dst3, zeros2)
    (x3,) = _tc4(a3a, a3b, g3, dinv, b3.reshape(1, D1),
                 ln3_g.reshape(1, D1), ln3_b.reshape(1, D1))
    return x3


# speedup vs baseline: 1.0385x; 1.0059x over previous
import functools

import jax
import jax.numpy as jnp
from jax import lax
from jax.experimental import pallas as pl
from jax.experimental.pallas import tpu as pltpu
from jax.experimental.pallas import tpu_sc as plsc

N = 10000
E = 320000
D1 = 128
D2 = 256

NC = 2
NS = 16
NW = NC * NS
EPT = E // NW
K = 50
NCH = EPT // K
SB = 40
NSB = NCH // SB
NRING = 4
KD = 125
NCHD = EPT // KD
RPT = 624
RPT_LAST = N - 15 * RPT

_mesh = plsc.VectorSubcoreMesh(core_axis_name="c", subcore_axis_name="s")


def _slab_copy(src, dst, s):
    r0 = s * RPT

    @pl.when(s < 15)
    def _():
        pltpu.sync_copy(src.at[pl.ds(r0, RPT)], dst.at[pl.ds(r0, RPT)])

    @pl.when(s == 15)
    def _():
        pltpu.sync_copy(src.at[pl.ds(15 * RPT, RPT_LAST)],
                        dst.at[pl.ds(15 * RPT, RPT_LAST)])


@functools.partial(
    pl.kernel,
    mesh=_mesh,
    out_type=[
        jax.ShapeDtypeStruct((N,), jnp.float32),
        jax.ShapeDtypeStruct((N,), jnp.float32),
    ],
    scratch_types=[
        pltpu.VMEM((NCHD, KD), jnp.int32),
        pltpu.VMEM((128,), jnp.float32),
        pltpu.VMEM_SHARED((N,), jnp.float32),
        pltpu.SemaphoreType.DMA,
    ],
)
def _sc_degree(dst_hbm, zeros_hbm, out0, out1, dst_all, ones_v, acc, sem):
    c = lax.axis_index("c")
    s = lax.axis_index("s")
    w = c * NS + s

    @pl.when(s == 0)
    def _():
        pltpu.sync_copy(zeros_hbm, acc)

    for i in range(8):
        ones_v[pl.ds(i * 16, 16)] = jnp.ones((16,), jnp.float32)
    pltpu.sync_copy(dst_hbm.at[w], dst_all)
    plsc.subcore_barrier()

    ones_src = ones_v.at[pl.ds(0, KD)]
    DEPTH = 8

    def start(j):
        pltpu.async_copy(ones_src, acc.at[dst_all.at[j]], sem, add=True)

    def drain_one():
        pltpu.make_async_copy(ones_src, acc.at[dst_all.at[0]], sem).wait()

    for j in range(DEPTH):
        start(j)

    def body(j, carry):
        drain_one()
        start(j + DEPTH)
        return carry

    lax.fori_loop(0, NCHD - DEPTH, body, 0)
    for _ in range(DEPTH):
        drain_one()
    plsc.subcore_barrier()

    @pl.when(s == 0)
    def _():
        @pl.when(c == 0)
        def _():
            pltpu.sync_copy(acc, out0)

        @pl.when(c == 1)
        def _():
            pltpu.sync_copy(acc, out1)


@functools.partial(
    pl.kernel,
    mesh=_mesh,
    out_type=[
        jax.ShapeDtypeStruct((N, D1), jnp.float32),
        jax.ShapeDtypeStruct((N, D1), jnp.float32),
    ],
    scratch_types=[
        pltpu.VMEM((SB, K), jnp.int32),
        pltpu.VMEM((SB, K), jnp.int32),
        pltpu.VMEM((SB, K), jnp.int32),
        pltpu.VMEM((SB, K), jnp.int32),
        pltpu.VMEM((K, D1), jnp.float32),
        pltpu.VMEM((K, D1), jnp.float32),
        pltpu.VMEM((K, D1), jnp.float32),
        pltpu.VMEM((K, D1), jnp.float32),
        pltpu.VMEM_SHARED((N, D1), jnp.float32),
        pltpu.SemaphoreType.DMA,
        pltpu.SemaphoreType.DMA,
        pltpu.SemaphoreType.DMA,
        pltpu.SemaphoreType.DMA,
        pltpu.SemaphoreType.DMA,
        pltpu.SemaphoreType.DMA,
        pltpu.SemaphoreType.DMA,
        pltpu.SemaphoreType.DMA,
        pltpu.SemaphoreType.DMA,
    ],
)
def _sc_scatter(g_hbm, src_hbm, dst_hbm, zeros_hbm, out0, out1,
                isrc0, idst0, isrc1, idst1, r0b, r1b, r2b, r3b, acc,
                gsem0, gsem1, gsem2, gsem3, ssem0, ssem1, ssem2, ssem3, isem):
    c = lax.axis_index("c")
    s = lax.axis_index("s")
    w = c * NS + s
    rows = [r0b, r1b, r2b, r3b]
    gsem = [gsem0, gsem1, gsem2, gsem3]
    ssem = [ssem0, ssem1, ssem2, ssem3]
    ibufs = [(isrc0, idst0), (isrc1, idst1)]

    _slab_copy(zeros_hbm, acc, s)
    pltpu.sync_copy(src_hbm.at[w, 0], isrc0)
    pltpu.sync_copy(dst_hbm.at[w, 0], idst0)
    plsc.subcore_barrier()

    for si in range(NSB):
        s_src, s_dst = ibufs[si & 1]
        if si + 1 < NSB:
            n_src, n_dst = ibufs[1 - (si & 1)]
            pltpu.async_copy(src_hbm.at[w, si + 1], n_src, isem)
            pltpu.async_copy(dst_hbm.at[w, si + 1], n_dst, isem)

        def gather_start(j, b):
            pltpu.async_copy(g_hbm.at[s_src.at[j]], rows[b], gsem[b])

        def gather_wait(b):
            pltpu.make_async_copy(g_hbm.at[s_src.at[0]], rows[b],
                                  gsem[b]).wait()

        def scatter_start(j, b):
            pltpu.async_copy(rows[b], acc.at[s_dst.at[j]], ssem[b], add=True)

        def scatter_wait(b):
            pltpu.make_async_copy(rows[b], acc.at[s_dst.at[0]],
                                  ssem[b]).wait()

        gather_start(0, 0)
        gather_start(1, 1)

        def body(t, carry):
            j0 = 4 * t
            for b in range(4):
                j = j0 + b
                gather_wait(b)
                scatter_start(j, b)
                jj = j + 2

                @pl.when(jnp.logical_and(jj < SB, jj - 4 >= 0))
                def _(b2=(b + 2) % 4, jj=jj):
                    scatter_wait(b2)
                    gather_start(jj, b2)

                @pl.when(jnp.logical_and(jj < SB, jj - 4 < 0))
                def _(b2=(b + 2) % 4, jj=jj):
                    gather_start(jj, b2)
            return carry

        lax.fori_loop(0, SB // 4, body, 0)
        for b in range(4):
            scatter_wait(b)
        if si + 1 < NSB:
            pltpu.make_async_copy(src_hbm.at[w, 0], n_src, isem).wait()
            pltpu.make_async_copy(dst_hbm.at[w, 0], n_dst, isem).wait()
    plsc.subcore_barrier()

    @pl.when(c == 0)
    def _():
        _slab_copy(acc, out0, s)

    @pl.when(c == 1)
    def _():
        _slab_copy(acc, out1, s)


R = 5000
GRID = N // R


def _row_spec(d):
    return pl.BlockSpec((R, d), lambda i: (i, 0))


def _full_spec(a, b):
    return pl.BlockSpec((a, b), lambda i: (0, 0))


def _ln(t, g, b):
    mu = jnp.mean(t, axis=-1, keepdims=True)
    var = jnp.mean((t - mu) ** 2, axis=-1, keepdims=True)
    return (t - mu) / jnp.sqrt(var + 1e-5) * g + b


def _tc1_body(x_ref, w1_ref, fw_ref, fb_ref, lg_ref, lb_ref, d0_ref, d1_ref,
              g1_ref, f1_ref, dinv_ref):
    x = x_ref[...]
    deg = d0_ref[...] + d1_ref[...] + 1.0
    dinv = lax.rsqrt(deg)
    dinv_ref[...] = dinv
    h1 = jnp.dot(x, w1_ref[...], preferred_element_type=jnp.float32)
    g1_ref[...] = h1 * dinv
    f = jnp.dot(x, fw_ref[...], preferred_element_type=jnp.float32) + fb_ref[...]
    f1_ref[...] = jnp.maximum(_ln(f, lg_ref[...], lb_ref[...]), 0.0)


_tc1 = pl.pallas_call(
    _tc1_body,
    grid=(GRID,),
    in_specs=[
        _row_spec(D1), _full_spec(D1, D1), _full_spec(D1, D1),
        _full_spec(1, D1), _full_spec(1, D1), _full_spec(1, D1),
        _row_spec(1), _row_spec(1),
    ],
    out_specs=[_row_spec(D1), _row_spec(D1), _row_spec(1)],
    out_shape=[
        jax.ShapeDtypeStruct((N, D1), jnp.float32),
        jax.ShapeDtypeStruct((N, D1), jnp.float32),
        jax.ShapeDtypeStruct((N, 1), jnp.float32),
    ],
)


def _tc2_body(a_ref, b_ref, g1_ref, dinv_ref, f1_ref, w2a_ref, w2b_ref, g2_ref):
    dinv = dinv_ref[...]
    x1 = jnp.maximum(dinv * (a_ref[...] + b_ref[...] + g1_ref[...]), 0.0)
    h2 = (jnp.dot(x1, w2a_ref[...], preferred_element_type=jnp.float32)
          + jnp.dot(f1_ref[...], w2b_ref[...], preferred_element_type=jnp.float32))
    g2_ref[...] = h2 * dinv


_tc2 = pl.pallas_call(
    _tc2_body,
    grid=(GRID,),
    in_specs=[
        _row_spec(D1), _row_spec(D1), _row_spec(D1), _row_spec(1),
        _row_spec(D1), _full_spec(D1, D1), _full_spec(D1, D1),
    ],
    out_specs=[_row_spec(D1)],
    out_shape=[jax.ShapeDtypeStruct((N, D1), jnp.float32)],
)


def _tc3_body(a_ref, b_ref, g2_ref, dinv_ref, b2_ref, lg_ref, lb_ref, w3_ref,
              g3_ref):
    dinv = dinv_ref[...]
    t = dinv * (a_ref[...] + b_ref[...] + g2_ref[...]) + b2_ref[...]
    x2 = jnp.maximum(_ln(t, lg_ref[...], lb_ref[...]), 0.0)
    g3_ref[...] = jnp.dot(x2, w3_ref[...], preferred_element_type=jnp.float32) * dinv


_tc3 = pl.pallas_call(
    _tc3_body,
    grid=(GRID,),
    in_specs=[
        _row_spec(D1), _row_spec(D1), _row_spec(D1), _row_spec(1),
        _full_spec(1, D1), _full_spec(1, D1), _full_spec(1, D1),
        _full_spec(D1, D1),
    ],
    out_specs=[_row_spec(D1)],
    out_shape=[jax.ShapeDtypeStruct((N, D1), jnp.float32)],
)


def _tc4_body(a_ref, b_ref, g3_ref, dinv_ref, b3_ref, lg_ref, lb_ref, x3_ref):
    dinv = dinv_ref[...]
    t = dinv * (a_ref[...] + b_ref[...] + g3_ref[...]) + b3_ref[...]
    x3_ref[...] = _ln(t, lg_ref[...], lb_ref[...])


_tc4 = pl.pallas_call(
    _tc4_body,
    grid=(GRID,),
    in_specs=[
        _row_spec(D1), _row_spec(D1), _row_spec(D1), _row_spec(1),
        _full_spec(1, D1), _full_spec(1, D1), _full_spec(1, D1),
    ],
    out_specs=[_row_spec(D1)],
    out_shape=[jax.ShapeDtypeStruct((N, D1), jnp.float32)],
)


def kernel(x, edge_index, W1, fc1_W, fc1_b, ln1_g, ln1_b, W2, b2, ln2_g, ln2_b,
           W3, b3, ln3_g, ln3_b):
    src3 = edge_index[0].reshape(NW, NSB, SB, K)
    dst3 = edge_index[1].reshape(NW, NSB, SB, K)
    dst3d = edge_index[1].reshape(NW, NCHD, KD)
    zeros1 = jnp.zeros((N,), jnp.float32)
    zeros2 = jnp.zeros((N, D1), jnp.float32)

    d0, d1 = _sc_degree(dst3d, zeros1)

    g1, f1, dinv = _tc1(
        x, W1, fc1_W, fc1_b.reshape(1, D1), ln1_g.reshape(1, D1),
        ln1_b.reshape(1, D1), d0.reshape(N, 1), d1.reshape(N, 1))

    a1a, a1b = _sc_scatter(g1, src3, dst3, zeros2)
    (g2,) = _tc2(a1a, a1b, g1, dinv, f1, W2[:D1], W2[D1:])

    a2a, a2b = _sc_scatter(g2, src3, dst3, zeros2)
    (g3,) = _tc3(a2a, a2b, g2, dinv, b2.reshape(1, D1),
                 ln2_g.reshape(1, D1), ln2_b.reshape(1, D1), W3)

    a3a, a3b = _sc_scatter(g3, src3, dst3, zeros2)
    (x3,) = _tc4(a3a, a3b, g3, dinv, b3.reshape(1, D1),
                 ln3_g.reshape(1, D1), ln3_b.reshape(1, D1))
    return x3
